# baseline (speedup 1.0000x reference)
import jax
import jax.numpy as jnp
from jax import lax
from jax.experimental import pallas as pl
from jax.experimental.pallas import tpu as pltpu
from jax.experimental.pallas import tpu_sc as plsc

_NUM_CLASS = 3
_NW = 32
_CH = 8192

_LP0 = 3.511021356705779e-06
_LP1 = 0.9997923620654495
_LP2 = -0.49697743071907685
_LP3 = 0.31458917398920905
_LP4 = -0.1887808235491981
_LP5 = 0.08172564529133709
_LP6 = -0.01720779923058697


def _sc_body(preds_hbm, labels_hbm, loss_hbm, pos_hbm, xbuf, labbuf, outv, sem):
    n = labels_hbm.shape[0]
    pt_per_w = n // _NW
    wid = lax.axis_index("s") * 2 + lax.axis_index("c")
    iota = lax.iota(jnp.int32, 16)
    perm = []
    for r in range(_NUM_CLASS):
        j = r * 16 + iota
        p = (j * 21846) >> 16
        perm.append((p, (j - p * 3 + 1).astype(jnp.int32)))

    def chunk_sums(c, carry):
        acc_l, acc_p = carry
        base = wid * pt_per_w + c * _CH
        pltpu.sync_copy(preds_hbm.at[pl.ds(base * _NUM_CLASS, _CH * _NUM_CLASS)], xbuf)
        pltpu.sync_copy(labels_hbm.at[pl.ds(base, _CH)], labbuf)

        def group(k, carry2):
            a_l, a_p = carry2
            lab16 = labbuf[pl.ds(k * 16, 16)]
            a_p = a_p + jnp.where(lab16 > 0, 1.0, 0.0)
            for r in range(_NUM_CLASS):
                x = xbuf[pl.ds(k * 48 + r * 16, 16)]
                idx_r, cls_r = perm[r]
                lab = lab16.at[idx_r].get(mode="promise_in_bounds")
                t = jnp.where(lab == cls_r, 1.0, 0.0)
                ax = jnp.abs(x)
                e = jnp.exp(-ax)
                sp = _LP0 + e * (
                    _LP1
                    + e * (_LP2 + e * (_LP3 + e * (_LP4 + e * (_LP5 + e * _LP6))))
                )
                s = jnp.where(x >= 0.0, 1.0, e) / (1.0 + e)
                bce = jnp.maximum(x, 0.0) - x * t + sp
                pt = t + s * (1.0 - 2.0 * t)
                aw = 0.75 - 0.5 * t
                a_l = a_l + aw * pt * pt * bce
            return a_l, a_p

        return lax.fori_loop(0, _CH // 16, group, (acc_l, acc_p))

    zero = jnp.zeros((16,), jnp.float32)
    acc_l, acc_p = lax.fori_loop(0, pt_per_w // _CH, chunk_sums, (zero, zero))
    outv[...] = acc_l
    pltpu.sync_copy(outv, loss_hbm.at[wid])
    outv[...] = acc_p
    pltpu.sync_copy(outv, pos_hbm.at[wid])
    del sem


def kernel(point_cls_preds, point_cls_labels):
    preds_flat = point_cls_preds.reshape(-1)
    mesh = plsc.VectorSubcoreMesh(core_axis_name="c", subcore_axis_name="s")
    sc_call = pl.kernel(
        _sc_body,
        out_type=[
            jax.ShapeDtypeStruct((_NW, 16), jnp.float32),
            jax.ShapeDtypeStruct((_NW, 16), jnp.float32),
        ],
        mesh=mesh,
        scratch_types=[
            pltpu.VMEM((_CH * _NUM_CLASS,), jnp.float32),
            pltpu.VMEM((_CH,), jnp.int32),
            pltpu.VMEM((16,), jnp.float32),
            pltpu.SemaphoreType.DMA,
        ],
    )
    loss_parts, pos_parts = sc_call(preds_flat, point_cls_labels)
    pos = jnp.sum(pos_parts)
    return jnp.sum(loss_parts) / jnp.maximum(pos, 1.0)


# speedup vs baseline: 13.9336x; 13.9336x over previous
import jax
import jax.numpy as jnp
from jax import lax
from jax.experimental import pallas as pl
from jax.experimental.pallas import tpu as pltpu
from jax.experimental.pallas import tpu_sc as plsc

_NUM_CLASS = 3
_NW = 32
_CH = 8192

_LP0 = 3.511021356705779e-06
_LP1 = 0.9997923620654495
_LP2 = -0.49697743071907685
_LP3 = 0.31458917398920905
_LP4 = -0.1887808235491981
_LP5 = 0.08172564529133709
_LP6 = -0.01720779923058697


def _sc_body(
    p0_hbm, p1_hbm, p2_hbm, labels_hbm, loss_hbm, pos_hbm, x0, x1, x2, labbuf, outv, sem
):
    n = labels_hbm.shape[0]
    pt_per_w = n // _NW
    wid = lax.axis_index("s") * 2 + lax.axis_index("c")
    planes = (p0_hbm, p1_hbm, p2_hbm)
    xbufs = (x0, x1, x2)

    def chunk_sums(c, carry):
        acc_l, acc_p = carry
        base = wid * pt_per_w + c * _CH
        for cls in range(_NUM_CLASS):
            pltpu.sync_copy(planes[cls].at[pl.ds(base, _CH)], xbufs[cls])
        pltpu.sync_copy(labels_hbm.at[pl.ds(base, _CH)], labbuf)

        def group(k, carry2):
            a_l, a_p = carry2
            lab16 = labbuf[pl.ds(k * 16, 16)]
            a_p = a_p + jnp.where(lab16 > 0, 1.0, 0.0)
            for cls in range(_NUM_CLASS):
                x = xbufs[cls][pl.ds(k * 16, 16)]
                t = jnp.where(lab16 == cls + 1, 1.0, 0.0)
                ax = jnp.abs(x)
                e = jnp.exp(-ax)
                sp = _LP0 + e * (
                    _LP1
                    + e * (_LP2 + e * (_LP3 + e * (_LP4 + e * (_LP5 + e * _LP6))))
                )
                s = jnp.where(x >= 0.0, 1.0, e) / (1.0 + e)
                bce = jnp.maximum(x, 0.0) - x * t + sp
                pt = t + s * (1.0 - 2.0 * t)
                aw = 0.75 - 0.5 * t
                a_l = a_l + aw * pt * pt * bce
            return a_l, a_p

        return lax.fori_loop(0, _CH // 16, group, (acc_l, acc_p))

    zero = jnp.zeros((16,), jnp.float32)
    acc_l, acc_p = lax.fori_loop(0, pt_per_w // _CH, chunk_sums, (zero, zero))
    outv[...] = acc_l
    pltpu.sync_copy(outv, loss_hbm.at[wid])
    outv[...] = acc_p
    pltpu.sync_copy(outv, pos_hbm.at[wid])
    del sem


def kernel(point_cls_preds, point_cls_labels):
    xt = point_cls_preds.T
    mesh = plsc.VectorSubcoreMesh(core_axis_name="c", subcore_axis_name="s")
    sc_call = pl.kernel(
        _sc_body,
        out_type=[
            jax.ShapeDtypeStruct((_NW, 16), jnp.float32),
            jax.ShapeDtypeStruct((_NW, 16), jnp.float32),
        ],
        mesh=mesh,
        scratch_types=[
            pltpu.VMEM((_CH,), jnp.float32),
            pltpu.VMEM((_CH,), jnp.float32),
            pltpu.VMEM((_CH,), jnp.float32),
            pltpu.VMEM((_CH,), jnp.int32),
            pltpu.VMEM((16,), jnp.float32),
            pltpu.SemaphoreType.DMA,
        ],
    )
    loss_parts, pos_parts = sc_call(xt[0], xt[1], xt[2], point_cls_labels)
    pos = jnp.sum(pos_parts)
    return jnp.sum(loss_parts) / jnp.maximum(pos, 1.0)


# speedup vs baseline: 18.3317x; 1.3156x over previous
import jax
import jax.numpy as jnp
from jax import lax
from jax.experimental import pallas as pl
from jax.experimental.pallas import tpu as pltpu
from jax.experimental.pallas import tpu_sc as plsc

_NUM_CLASS = 3
_NW = 32
_CH = 8192

_LP0 = 3.511021356705779e-06
_LP1 = 0.9997923620654495
_LP2 = -0.49697743071907685
_LP3 = 0.31458917398920905
_LP4 = -0.1887808235491981
_LP5 = 0.08172564529133709
_LP6 = -0.01720779923058697


def _sc_body(xt_hbm, labels_hbm, loss_hbm, pos_hbm, xbuf, labbuf, outv, sem):
    n = labels_hbm.shape[0]
    pt_per_w = n // _NW
    wid = lax.axis_index("s") * 2 + lax.axis_index("c")

    def chunk_sums(c, carry):
        acc_l, acc_p = carry
        base = wid * pt_per_w + c * _CH
        pltpu.sync_copy(xt_hbm.at[pl.ds(0, _NUM_CLASS), pl.ds(base, _CH)], xbuf)
        pltpu.sync_copy(labels_hbm.at[pl.ds(base, _CH)], labbuf)

        def group(k, carry2):
            a_l, a_p = carry2
            lab16 = labbuf[pl.ds(k * 16, 16)]
            a_p = a_p + jnp.where(lab16 > 0, 1.0, 0.0)
            for cls in range(_NUM_CLASS):
                x = xbuf[cls, pl.ds(k * 16, 16)]
                t = jnp.where(lab16 == cls + 1, 1.0, 0.0)
                ax = jnp.abs(x)
                e = jnp.exp(-ax)
                sp = _LP0 + e * (
                    _LP1
                    + e * (_LP2 + e * (_LP3 + e * (_LP4 + e * (_LP5 + e * _LP6))))
                )
                s = jnp.where(x >= 0.0, 1.0, e) / (1.0 + e)
                bce = jnp.maximum(x, 0.0) - x * t + sp
                pt = t + s * (1.0 - 2.0 * t)
                aw = 0.75 - 0.5 * t
                a_l = a_l + aw * pt * pt * bce
            return a_l, a_p

        return lax.fori_loop(0, _CH // 16, group, (acc_l, acc_p))

    zero = jnp.zeros((16,), jnp.float32)
    acc_l, acc_p = lax.fori_loop(0, pt_per_w // _CH, chunk_sums, (zero, zero))
    outv[...] = acc_l
    pltpu.sync_copy(outv, loss_hbm.at[wid])
    outv[...] = acc_p
    pltpu.sync_copy(outv, pos_hbm.at[wid])
    del sem


def kernel(point_cls_preds, point_cls_labels):
    xt = point_cls_preds.T
    mesh = plsc.VectorSubcoreMesh(core_axis_name="c", subcore_axis_name="s")
    sc_call = pl.kernel(
        _sc_body,
        out_type=[
            jax.ShapeDtypeStruct((_NW, 16), jnp.float32),
            jax.ShapeDtypeStruct((_NW, 16), jnp.float32),
        ],
        mesh=mesh,
        scratch_types=[
            pltpu.VMEM((_NUM_CLASS, _CH), jnp.float32),
            pltpu.VMEM((_CH,), jnp.int32),
            pltpu.VMEM((16,), jnp.float32),
            pltpu.SemaphoreType.DMA,
        ],
    )
    loss_parts, pos_parts = sc_call(xt, point_cls_labels)
    pos = jnp.sum(pos_parts)
    return jnp.sum(loss_parts) / jnp.maximum(pos, 1.0)


# speedup vs baseline: 18.4248x; 1.0051x over previous
import jax
import jax.numpy as jnp
from jax import lax
from jax.experimental import pallas as pl
from jax.experimental.pallas import tpu as pltpu
from jax.experimental.pallas import tpu_sc as plsc

_NUM_CLASS = 3
_NW = 32
_CH = 8192

_LP0 = 3.511021356705779e-06
_LP1 = 0.9997923620654495
_LP2 = -0.49697743071907685
_LP3 = 0.31458917398920905
_LP4 = -0.1887808235491981
_LP5 = 0.08172564529133709
_LP6 = -0.01720779923058697


def _sc_body(xt_hbm, labels_hbm, loss_hbm, pos_hbm, xbuf, labbuf, outv, sem):
    n = labels_hbm.shape[0]
    pt_per_w = n // _NW
    wid = lax.axis_index("s") * 2 + lax.axis_index("c")

    def chunk_sums(c, carry):
        acc_l, acc_p = carry
        base = wid * pt_per_w + c * _CH
        pltpu.sync_copy(xt_hbm.at[pl.ds(0, _NUM_CLASS), pl.ds(base, _CH)], xbuf)
        pltpu.sync_copy(labels_hbm.at[pl.ds(base, _CH)], labbuf)

        @plsc.parallel_loop(0, _CH // 16, carry=(acc_l, acc_p), unroll=4)
        def group(k, carry2):
            a_l, a_p = carry2
            lab16 = labbuf[pl.ds(k * 16, 16)]
            a_p = a_p + jnp.where(lab16 > 0, 1.0, 0.0)
            for cls in range(_NUM_CLASS):
                x = xbuf[cls, pl.ds(k * 16, 16)]
                t = jnp.where(lab16 == cls + 1, 1.0, 0.0)
                u = 1.0 - 2.0 * t
                ax = jnp.abs(x)
                e = jnp.exp(-ax)
                sp = _LP0 + e * (
                    _LP1
                    + e * (_LP2 + e * (_LP3 + e * (_LP4 + e * (_LP5 + e * _LP6))))
                )
                s = jnp.where(x >= 0.0, 1.0, e) / (1.0 + e)
                bce = jnp.maximum(x * u, 0.0) + sp
                pt = t + s * u
                aw = 0.75 - 0.5 * t
                a_l = a_l + aw * pt * pt * bce
            return a_l, a_p

        return group

    zero = jnp.zeros((16,), jnp.float32)
    acc_l, acc_p = lax.fori_loop(0, pt_per_w // _CH, chunk_sums, (zero, zero))
    outv[...] = acc_l
    pltpu.sync_copy(outv, loss_hbm.at[wid])
    outv[...] = acc_p
    pltpu.sync_copy(outv, pos_hbm.at[wid])
    del sem


def kernel(point_cls_preds, point_cls_labels):
    xt = point_cls_preds.T
    mesh = plsc.VectorSubcoreMesh(core_axis_name="c", subcore_axis_name="s")
    sc_call = pl.kernel(
        _sc_body,
        out_type=[
            jax.ShapeDtypeStruct((_NW, 16), jnp.float32),
            jax.ShapeDtypeStruct((_NW, 16), jnp.float32),
        ],
        mesh=mesh,
        scratch_types=[
            pltpu.VMEM((_NUM_CLASS, _CH), jnp.float32),
            pltpu.VMEM((_CH,), jnp.int32),
            pltpu.VMEM((16,), jnp.float32),
            pltpu.SemaphoreType.DMA,
        ],
    )
    loss_parts, pos_parts = sc_call(xt, point_cls_labels)
    pos = jnp.sum(pos_parts)
    return jnp.sum(loss_parts) / jnp.maximum(pos, 1.0)


# speedup vs baseline: 18.6029x; 1.0097x over previous
import jax
import jax.numpy as jnp
from jax import lax
from jax.experimental import pallas as pl
from jax.experimental.pallas import tpu as pltpu
from jax.experimental.pallas import tpu_sc as plsc

_NUM_CLASS = 3
_NW = 32
_CH = 8192

_LP0 = 3.511021356705779e-06
_LP1 = 0.9997923620654495
_LP2 = -0.49697743071907685
_LP3 = 0.31458917398920905
_LP4 = -0.1887808235491981
_LP5 = 0.08172564529133709
_LP6 = -0.01720779923058697


def _sc_body(xt_hbm, labels_hbm, loss_hbm, pos_hbm, xbuf, labbuf, outv, sem):
    n = labels_hbm.shape[0]
    pt_per_w = n // _NW
    wid = lax.axis_index("s") * 2 + lax.axis_index("c")

    def chunk_sums(c, carry):
        base = wid * pt_per_w + c * _CH
        pltpu.sync_copy(xt_hbm.at[pl.ds(0, _NUM_CLASS), pl.ds(base, _CH)], xbuf)
        pltpu.sync_copy(labels_hbm.at[pl.ds(base, _CH)], labbuf)

        @plsc.parallel_loop(0, _CH // 16, carry=carry, unroll=4)
        def group(k, carry2):
            a0, a1, a2, a_p = carry2
            accs = [a0, a1, a2]
            lab16 = labbuf[pl.ds(k * 16, 16)]
            a_p = a_p + jnp.where(lab16 > 0, 1.0, 0.0)
            for cls in range(_NUM_CLASS):
                x = xbuf[cls, pl.ds(k * 16, 16)]
                t = jnp.where(lab16 == cls + 1, 1.0, 0.0)
                u = 1.0 - 2.0 * t
                ax = jnp.abs(x)
                e = jnp.exp(-ax)
                sp = _LP0 + e * (
                    _LP1
                    + e * (_LP2 + e * (_LP3 + e * (_LP4 + e * (_LP5 + e * _LP6))))
                )
                s = jnp.where(x >= 0.0, 1.0, e) / (1.0 + e)
                bce = jnp.maximum(x * u, 0.0) + sp
                pt = t + s * u
                aw = 0.75 - 0.5 * t
                accs[cls] = accs[cls] + aw * pt * pt * bce
            return accs[0], accs[1], accs[2], a_p

        return group

    zero = jnp.zeros((16,), jnp.float32)
    a0, a1, a2, acc_p = lax.fori_loop(
        0, pt_per_w // _CH, chunk_sums, (zero, zero, zero, zero)
    )
    acc_l = (a0 + a1) + a2
    outv[...] = acc_l
    pltpu.sync_copy(outv, loss_hbm.at[wid])
    outv[...] = acc_p
    pltpu.sync_copy(outv, pos_hbm.at[wid])
    del sem


def kernel(point_cls_preds, point_cls_labels):
    xt = point_cls_preds.T
    mesh = plsc.VectorSubcoreMesh(core_axis_name="c", subcore_axis_name="s")
    sc_call = pl.kernel(
        _sc_body,
        out_type=[
            jax.ShapeDtypeStruct((_NW, 16), jnp.float32),
            jax.ShapeDtypeStruct((_NW, 16), jnp.float32),
        ],
        mesh=mesh,
        scratch_types=[
            pltpu.VMEM((_NUM_CLASS, _CH), jnp.float32),
            pltpu.VMEM((_CH,), jnp.int32),
            pltpu.VMEM((16,), jnp.float32),
            pltpu.SemaphoreType.DMA,
        ],
    )
    loss_parts, pos_parts = sc_call(xt, point_cls_labels)
    pos = jnp.sum(pos_parts)
    return jnp.sum(loss_parts) / jnp.maximum(pos, 1.0)


# speedup vs baseline: 23.2772x; 1.2513x over previous
import jax
import jax.numpy as jnp
from jax import lax
from jax.experimental import pallas as pl
from jax.experimental.pallas import tpu as pltpu
from jax.experimental.pallas import tpu_sc as plsc

_NUM_CLASS = 3
_NW = 32
_CH = 8192

_LP = (
    2.213365940550993e-05,
    0.9990101957178754,
    -0.4891557228193072,
    0.2833022757650565,
    -0.13011784776014965,
    0.030102226625878577,
)


def _sc_body(
    xt_hbm, labels_hbm, loss_hbm, pos_hbm, xb0, xb1, lb0, lb1, outv, sem0, sem1
):
    n = labels_hbm.shape[0]
    pt_per_w = n // _NW
    n_chunks = pt_per_w // _CH
    wid = lax.axis_index("s") * 2 + lax.axis_index("c")
    xbufs = (xb0, xb1)
    lbufs = (lb0, lb1)
    sems = (sem0, sem1)

    def issue(c, b):
        base = wid * pt_per_w + c * _CH
        hx = pltpu.async_copy(
            xt_hbm.at[pl.ds(0, _NUM_CLASS), pl.ds(base, _CH)], xbufs[b], sems[b]
        )
        hl = pltpu.async_copy(labels_hbm.at[pl.ds(base, _CH)], lbufs[b], sems[b])
        return hx, hl

    carry = (jnp.zeros((16,), jnp.float32),) * 4
    pending = issue(0, 0)
    for c in range(n_chunks):
        b = c % 2
        pending[0].wait()
        pending[1].wait()
        if c + 1 < n_chunks:
            pending = issue(c + 1, (c + 1) % 2)
        xbuf = xbufs[b]
        labbuf = lbufs[b]

        @plsc.parallel_loop(0, _CH // 16, carry=carry, unroll=4)
        def group(k, carry2):
            a0, a1, a2, a_p = carry2
            accs = [a0, a1, a2]
            lab16 = labbuf[pl.ds(k * 16, 16)]
            a_p = a_p + jnp.where(lab16 > 0, 1.0, 0.0)
            for cls in range(_NUM_CLASS):
                x = xbuf[cls, pl.ds(k * 16, 16)]
                hot = lab16 == cls + 1
                u = jnp.where(hot, -1.0, 1.0)
                aw = jnp.where(hot, 0.25, 0.75)
                y = x * u
                e = jnp.exp(-jnp.abs(x))
                sp = _LP[0] + e * (
                    _LP[1] + e * (_LP[2] + e * (_LP[3] + e * (_LP[4] + e * _LP[5])))
                )
                bce = jnp.maximum(y, 0.0) + sp
                pt = jnp.where(y >= 0.0, 1.0, e) / (1.0 + e)
                accs[cls] = accs[cls] + (aw * (pt * pt)) * bce
            return accs[0], accs[1], accs[2], a_p

        carry = group

    a0, a1, a2, acc_p = carry
    outv[...] = (a0 + a1) + a2
    pltpu.sync_copy(outv, loss_hbm.at[wid])
    outv[...] = acc_p
    pltpu.sync_copy(outv, pos_hbm.at[wid])


def kernel(point_cls_preds, point_cls_labels):
    xt = point_cls_preds.T
    mesh = plsc.VectorSubcoreMesh(core_axis_name="c", subcore_axis_name="s")
    sc_call = pl.kernel(
        _sc_body,
        out_type=[
            jax.ShapeDtypeStruct((_NW, 16), jnp.float32),
            jax.ShapeDtypeStruct((_NW, 16), jnp.float32),
        ],
        mesh=mesh,
        scratch_types=[
            pltpu.VMEM((_NUM_CLASS, _CH), jnp.float32),
            pltpu.VMEM((_NUM_CLASS, _CH), jnp.float32),
            pltpu.VMEM((_CH,), jnp.int32),
            pltpu.VMEM((_CH,), jnp.int32),
            pltpu.VMEM((16,), jnp.float32),
            pltpu.SemaphoreType.DMA,
            pltpu.SemaphoreType.DMA,
        ],
    )
    loss_parts, pos_parts = sc_call(xt, point_cls_labels)
    pos = jnp.sum(pos_parts)
    return jnp.sum(loss_parts) / jnp.maximum(pos, 1.0)


# speedup vs baseline: 23.2977x; 1.0009x over previous
import jax
import jax.numpy as jnp
from jax import lax
from jax.experimental import pallas as pl
from jax.experimental.pallas import tpu as pltpu
from jax.experimental.pallas import tpu_sc as plsc

_NUM_CLASS = 3
_NW = 32
_CH = 8192

_LP = (
    2.213365940550993e-05,
    0.9990101957178754,
    -0.4891557228193072,
    0.2833022757650565,
    -0.13011784776014965,
    0.030102226625878577,
)


def _sc_body(
    xt_hbm, labels_hbm, loss_hbm, pos_hbm, xb0, xb1, lb0, lb1, outv, sem0, sem1
):
    n = labels_hbm.shape[0]
    pt_per_w = n // _NW
    n_chunks = pt_per_w // _CH
    wid = lax.axis_index("s") * 2 + lax.axis_index("c")
    xbufs = (xb0, xb1)
    lbufs = (lb0, lb1)
    sems = (sem0, sem1)

    def issue(c, b):
        base = wid * pt_per_w + c * _CH
        hx = pltpu.async_copy(
            xt_hbm.at[pl.ds(0, _NUM_CLASS), pl.ds(base, _CH)], xbufs[b], sems[b]
        )
        hl = pltpu.async_copy(labels_hbm.at[pl.ds(base, _CH)], lbufs[b], sems[b])
        return hx, hl

    carry = (jnp.zeros((16,), jnp.float32),) * 4
    pending = issue(0, 0)
    for c in range(n_chunks):
        b = c % 2
        pending[0].wait()
        pending[1].wait()
        if c + 1 < n_chunks:
            pending = issue(c + 1, (c + 1) % 2)
        xbuf = xbufs[b]
        labbuf = lbufs[b]

        @plsc.parallel_loop(0, _CH // 16, carry=carry, unroll=8)
        def group(k, carry2):
            a0, a1, a2, a_p = carry2
            accs = [a0, a1, a2]
            lab16 = labbuf[pl.ds(k * 16, 16)]
            a_p = a_p + jnp.where(lab16 > 0, 1.0, 0.0)
            for cls in range(_NUM_CLASS):
                x = xbuf[cls, pl.ds(k * 16, 16)]
                hot = lab16 == cls + 1
                u = jnp.where(hot, -1.0, 1.0)
                aw = jnp.where(hot, 0.25, 0.75)
                y = x * u
                e = jnp.exp(-jnp.abs(x))
                sp = _LP[0] + e * (
                    _LP[1] + e * (_LP[2] + e * (_LP[3] + e * (_LP[4] + e * _LP[5])))
                )
                bce = jnp.maximum(y, 0.0) + sp
                pt = jnp.where(y >= 0.0, 1.0, e) / (1.0 + e)
                accs[cls] = accs[cls] + (aw * (pt * pt)) * bce
            return accs[0], accs[1], accs[2], a_p

        carry = group

    a0, a1, a2, acc_p = carry
    outv[...] = (a0 + a1) + a2
    pltpu.sync_copy(outv, loss_hbm.at[wid])
    outv[...] = acc_p
    pltpu.sync_copy(outv, pos_hbm.at[wid])


def kernel(point_cls_preds, point_cls_labels):
    xt = point_cls_preds.T
    mesh = plsc.VectorSubcoreMesh(core_axis_name="c", subcore_axis_name="s")
    sc_call = pl.kernel(
        _sc_body,
        out_type=[
            jax.ShapeDtypeStruct((_NW, 16), jnp.float32),
            jax.ShapeDtypeStruct((_NW, 16), jnp.float32),
        ],
        mesh=mesh,
        scratch_types=[
            pltpu.VMEM((_NUM_CLASS, _CH), jnp.float32),
            pltpu.VMEM((_NUM_CLASS, _CH), jnp.float32),
            pltpu.VMEM((_CH,), jnp.int32),
            pltpu.VMEM((_CH,), jnp.int32),
            pltpu.VMEM((16,), jnp.float32),
            pltpu.SemaphoreType.DMA,
            pltpu.SemaphoreType.DMA,
        ],
    )
    loss_parts, pos_parts = sc_call(xt, point_cls_labels)
    pos = jnp.sum(pos_parts)
    return jnp.sum(loss_parts) / jnp.maximum(pos, 1.0)
